# baseline (speedup 1.0000x reference)
import functools

import jax
import jax.numpy as jnp
from jax import lax
from jax.experimental import pallas as pl
from jax.experimental.pallas import tpu as pltpu
from jax.experimental.pallas import tpu_sc as plsc

B = 4096
L = 200
H = 32
NUM_LABELS = 2
V = 1000000
SLOT = 8

NC = 2
NS = 16
NW = NC * NS
CPW = B // NW

PROJ_BLK = 32768
R2 = PROJ_BLK // 128
OR = SLOT * PROJ_BLK // 128
NBLK = pl.cdiv(V, PROJ_BLK)
VP = NBLK * PROJ_BLK


def _proj_body(tbl_ref, w_ref, u8_ref):
    ut = lax.dot_general(
        w_ref[...], tbl_ref[...], (((1,), (0,)), ((), ())),
        preferred_element_type=jnp.float32,
    )
    i0 = pl.program_id(0) * PROJ_BLK
    jc = lax.broadcasted_iota(jnp.int32, (NUM_LABELS, PROJ_BLK), 1)
    ut = jnp.where(jc < V - i0, ut, 0.0)
    u0r = ut[0].reshape(R2, 128)
    u1r = ut[1].reshape(R2, 128)
    cc = lax.broadcasted_iota(jnp.int32, (128, 128), 0)
    col = lax.broadcasted_iota(jnp.int32, (128, 128), 1)
    rr = lax.broadcasted_iota(jnp.int32, (OR, R2), 0)
    c2 = lax.broadcasted_iota(jnp.int32, (OR, R2), 1)
    acc = jnp.zeros((OR, 128), jnp.float32)
    for p in range(8):
        ap = ((col % 8 == 0) & (cc == 16 * p + col // 8)).astype(jnp.float32)
        bp = ((col % 8 == 1) & (cc == 16 * p + col // 8)).astype(jnp.float32)
        op = (lax.dot_general(u0r, ap, (((1,), (0,)), ((), ())),
                              preferred_element_type=jnp.float32)
              + lax.dot_general(u1r, bp, (((1,), (0,)), ((), ())),
                                preferred_element_type=jnp.float32))
        ep = (rr == 8 * c2 + p).astype(jnp.float32)
        acc = acc + lax.dot_general(ep, op, (((1,), (0,)), ((), ())),
                                    preferred_element_type=jnp.float32)
    u8_ref[...] = acc


@jax.jit
def _project(table_t, w):
    return pl.pallas_call(
        _proj_body,
        grid=(NBLK,),
        in_specs=[
            pl.BlockSpec((H, PROJ_BLK), lambda i: (0, i)),
            pl.BlockSpec((NUM_LABELS, H), lambda i: (0, 0)),
        ],
        out_specs=pl.BlockSpec((OR, 128), lambda i: (i, 0)),
        out_shape=jax.ShapeDtypeStruct((NBLK * OR, 128), jnp.float32),
    )(table_t, w)


GROUP_L = 25
NGRP = L // GROUP_L


def _sc_gather_body(ids_hbm, u8_hbm, a2_hbm, idx_v, ga, gb, gsa, gsb, osa, osb):
    wid = lax.axis_index("s") * NC + lax.axis_index("c")
    col0 = wid * CPW
    pltpu.sync_copy(ids_hbm.at[:, pl.ds(col0, CPW)], idx_v)

    bufs = (ga, gb)
    gsems = (gsa, gsb)
    osems = (osa, osb)

    def a2_slc(grp):
        return a2_hbm.at[pl.ds(grp * GROUP_L, GROUP_L), pl.ds(col0, CPW)]

    def issue_group(grp, buf, gsem):
        base_j = grp * GROUP_L

        def issue(j, _):
            pltpu.make_async_copy(
                u8_hbm.at[idx_v.at[base_j + j]], buf.at[j], gsem).start()
            return 0

        lax.fori_loop(0, GROUP_L, issue, 0)

    def drain_group(buf, gsem):
        dummy = u8_hbm.at[pl.ds(0, CPW)]

        def drain(j, _):
            pltpu.make_async_copy(dummy, buf.at[j], gsem).wait()
            return 0

        lax.fori_loop(0, GROUP_L, drain, 0)

    issue_group(0, bufs[0], gsems[0])
    for grp in range(NGRP):
        bsel = grp % 2
        nxt = grp + 1
        if nxt < NGRP:
            nb = nxt % 2
            if nxt >= 2:
                pltpu.make_async_copy(
                    bufs[nb].at[:, :, pl.ds(0, NUM_LABELS)],
                    a2_slc(nxt - 2), osems[nb]).wait()
            issue_group(nxt, bufs[nb], gsems[nb])
        drain_group(bufs[bsel], gsems[bsel])
        pltpu.make_async_copy(
            bufs[bsel].at[:, :, pl.ds(0, NUM_LABELS)],
            a2_slc(grp), osems[bsel]).start()
    pltpu.make_async_copy(
        bufs[0].at[:, :, pl.ds(0, NUM_LABELS)], a2_slc(NGRP - 2), osems[0]).wait()
    pltpu.make_async_copy(
        bufs[1].at[:, :, pl.ds(0, NUM_LABELS)], a2_slc(NGRP - 1), osems[1]).wait()


@jax.jit
def _sc_gather(ids_t, u8_tbl):
    mesh = plsc.VectorSubcoreMesh(
        core_axis_name="c", subcore_axis_name="s", num_cores=NC, num_subcores=NS
    )
    return pl.kernel(
        _sc_gather_body,
        out_type=jax.ShapeDtypeStruct((L, B, NUM_LABELS), jnp.float32),
        mesh=mesh,
        compiler_params=pltpu.CompilerParams(use_tc_tiling_on_sc=False),
        scratch_types=[
            pltpu.VMEM((L, CPW), jnp.int32),
            pltpu.VMEM((GROUP_L, CPW, SLOT), jnp.float32),
            pltpu.VMEM((GROUP_L, CPW, SLOT), jnp.float32),
            pltpu.SemaphoreType.DMA,
            pltpu.SemaphoreType.DMA,
            pltpu.SemaphoreType.DMA,
            pltpu.SemaphoreType.DMA,
        ],
    )(ids_t, u8_tbl)


def _head_body(a2_ref, mrep_ref, labels_ref, b_ref, loss_ref, logits_ref):
    m3 = mrep_ref[...]
    s = jnp.sum(a2_ref[...] * m3, axis=0)
    denom = jnp.maximum(jnp.sum(m3, axis=0), 1.0)
    lane = lax.broadcasted_iota(jnp.int32, (B // 64, 128), 1)
    bias = jnp.where(lane % 2 == 0, b_ref[0, 0], b_ref[0, 1])
    logits_il = s / denom + bias
    logits_ref[...] = logits_il
    i0 = lax.broadcasted_iota(jnp.int32, (128, 128), 0)
    i1 = lax.broadcasted_iota(jnp.int32, (128, 128), 1)
    pq = ((i0 % 2) * 64 + i0 // 2 == i1).astype(jnp.float32)
    q = lax.dot_general(logits_il, pq, (((1,), (0,)), ((), ())),
                        preferred_element_type=jnp.float32)
    l0 = q[:, 0:64]
    l1 = q[:, 64:128]
    mx = jnp.maximum(l0, l1)
    lse = mx + jnp.log(jnp.exp(l0 - mx) + jnp.exp(l1 - mx))
    sel = jnp.where(labels_ref[...] == 0, l0, l1)
    loss_ref[...] = jnp.sum(lse - sel, axis=(0, 1), keepdims=True) * (1.0 / B)


@jax.jit
def _head(a2_3d, mrep_3d, labels_2d, b2d):
    return pl.pallas_call(
        _head_body,
        out_shape=(
            jax.ShapeDtypeStruct((1, 1), jnp.float32),
            jax.ShapeDtypeStruct((B // 64, 128), jnp.float32),
        ),
    )(a2_3d, mrep_3d, labels_2d, b2d)


def kernel(input_ids, attention_mask, labels, emb_table, W, b):
    table_t = emb_table.T
    u8 = _project(table_t, W)
    u8_tbl = u8.reshape(VP, SLOT)
    ids_t = input_ids.T.astype(jnp.int32)
    a2 = _sc_gather(ids_t, u8_tbl)
    mrep = jnp.repeat(attention_mask.T, NUM_LABELS, axis=1)
    loss11, logits_il = _head(
        a2.reshape(L, B // 64, 128),
        mrep.reshape(L, B // 64, 128),
        labels.reshape(B // 64, 64).astype(jnp.int32),
        b.reshape(1, NUM_LABELS),
    )
    return loss11[0, 0], logits_il.reshape(B, NUM_LABELS)


# speedup vs baseline: 16.5696x; 16.5696x over previous
import functools

import jax
import jax.numpy as jnp
from jax import lax
from jax.experimental import pallas as pl
from jax.experimental.pallas import tpu as pltpu
from jax.experimental.pallas import tpu_sc as plsc

B = 4096
L = 200
H = 32
NUM_LABELS = 2
V = 1000000

NC = 2
NS = 16
NW = NC * NS
CPW = B // NW

PROJ_BLK = 65536

SUBL = B // 128


def _proj_body(tbl_ref, w_ref, u0_ref, u1_ref):
    ut = lax.dot_general(
        w_ref[...], tbl_ref[...], (((1,), (0,)), ((), ())),
        preferred_element_type=jnp.float32,
    )
    u0_ref[...] = ut[0]
    u1_ref[...] = ut[1]


@jax.jit
def _project(table_t, w):
    grid = (pl.cdiv(V, PROJ_BLK),)
    return pl.pallas_call(
        _proj_body,
        grid=grid,
        in_specs=[
            pl.BlockSpec((H, PROJ_BLK), lambda i: (0, i)),
            pl.BlockSpec((NUM_LABELS, H), lambda i: (0, 0)),
        ],
        out_specs=[
            pl.BlockSpec((PROJ_BLK,), lambda i: (i,)),
            pl.BlockSpec((PROJ_BLK,), lambda i: (i,)),
        ],
        out_shape=[
            jax.ShapeDtypeStruct((V,), jnp.float32),
            jax.ShapeDtypeStruct((V,), jnp.float32),
        ],
    )(table_t, w)


def _sc_gather_body(ids_hbm, u0_hbm, u1_hbm, a0_hbm, a1_hbm,
                    idx_v, g0, g1, sem0, sem1):
    wid = lax.axis_index("s") * NC + lax.axis_index("c")
    col0 = wid * CPW
    pltpu.sync_copy(ids_hbm.at[:, pl.ds(col0, CPW)], idx_v)

    def issue(j, _):
        pltpu.make_async_copy(u0_hbm.at[idx_v.at[j]], g0.at[j], sem0).start()
        pltpu.make_async_copy(u1_hbm.at[idx_v.at[j]], g1.at[j], sem1).start()
        return 0

    lax.fori_loop(0, L, issue, 0)
    pltpu.make_async_copy(a0_hbm.at[:, pl.ds(col0, CPW)], g0, sem0).wait()
    pltpu.make_async_copy(a1_hbm.at[:, pl.ds(col0, CPW)], g1, sem1).wait()
    pltpu.sync_copy(g0, a0_hbm.at[:, pl.ds(col0, CPW)])
    pltpu.sync_copy(g1, a1_hbm.at[:, pl.ds(col0, CPW)])


@jax.jit
def _sc_gather(ids_t, u0, u1):
    mesh = plsc.VectorSubcoreMesh(
        core_axis_name="c", subcore_axis_name="s", num_cores=NC, num_subcores=NS
    )
    return pl.kernel(
        _sc_gather_body,
        out_type=(
            jax.ShapeDtypeStruct((L, B), jnp.float32),
            jax.ShapeDtypeStruct((L, B), jnp.float32),
        ),
        mesh=mesh,
        compiler_params=pltpu.CompilerParams(use_tc_tiling_on_sc=False),
        scratch_types=[
            pltpu.VMEM((L, CPW), jnp.int32),
            pltpu.VMEM((L, CPW), jnp.float32),
            pltpu.VMEM((L, CPW), jnp.float32),
            pltpu.SemaphoreType.DMA,
            pltpu.SemaphoreType.DMA,
        ],
    )(ids_t, u0, u1)


def _head_body(a0_ref, a1_ref, mask_ref, labels_ref, b_ref,
               loss_ref, l0_ref, l1_ref):
    m = mask_ref[...]
    denom = jnp.maximum(jnp.sum(m, axis=0), 1.0)
    l0 = jnp.sum(a0_ref[...] * m, axis=0) / denom + b_ref[0, 0]
    l1 = jnp.sum(a1_ref[...] * m, axis=0) / denom + b_ref[0, 1]
    mx = jnp.maximum(l0, l1)
    lse = mx + jnp.log(jnp.exp(l0 - mx) + jnp.exp(l1 - mx))
    sel = jnp.where(labels_ref[...] == 0, l0, l1)
    loss_ref[...] = jnp.sum(lse - sel, axis=(0, 1), keepdims=True) * (1.0 / B)
    l0_ref[...] = l0
    l1_ref[...] = l1


@jax.jit
def _head(a0_3d, a1_3d, mask_3d, labels_2d, b2d):
    return pl.pallas_call(
        _head_body,
        out_shape=(
            jax.ShapeDtypeStruct((1, 1), jnp.float32),
            jax.ShapeDtypeStruct((SUBL, 128), jnp.float32),
            jax.ShapeDtypeStruct((SUBL, 128), jnp.float32),
        ),
    )(a0_3d, a1_3d, mask_3d, labels_2d, b2d)


def kernel(input_ids, attention_mask, labels, emb_table, W, b):
    table_t = emb_table.T
    u0, u1 = _project(table_t, W)
    ids_t = input_ids.T.astype(jnp.int32)
    a0, a1 = _sc_gather(ids_t, u0, u1)
    loss11, l0, l1 = _head(
        a0.reshape(L, SUBL, 128),
        a1.reshape(L, SUBL, 128),
        attention_mask.T.reshape(L, SUBL, 128),
        labels.reshape(SUBL, 128).astype(jnp.int32),
        b.reshape(1, NUM_LABELS),
    )
    logits = jnp.stack([l0.reshape(B), l1.reshape(B)], axis=-1)
    return loss11[0, 0], logits
